# baseline (speedup 1.0000x reference)
import functools

import jax
import jax.numpy as jnp
from jax import lax
from jax.experimental import pallas as pl
from jax.experimental.pallas import tpu as pltpu
from jax.experimental.pallas import tpu_sc as plsc

N = 10000
E = 320000
D = 128

NC = 2
NS = 16
NW = NC * NS
CHUNK = 128
NCHUNK = 80
EPW = NCHUNK * CHUNK
E_PAD = EPW * NW
H_ROWS = N + 16
ACC_ROWS = 10112
RPS = ACC_ROWS // NS
PACK = 1 << 14
PAD_EDGE = N * PACK + N
C0 = 48
C1 = 2 * NCHUNK - C0
CMAX = max(C0, C1)

_mesh = plsc.VectorSubcoreMesh(core_axis_name="c", subcore_axis_name="s")


def _store_const(buf, rows, value, dtype=jnp.float32):
    vv = jnp.full((16,), value, dtype)

    def body(i, _):
        for j in range(D // 16):
            buf[i, pl.ds(j * 16, 16)] = vv
        return 0

    lax.fori_loop(0, rows, body, 0)


def _zero_stripe(zsrc, acc, s, sem):
    r0 = s * RPS
    pltpu.async_copy(zsrc, acc.at[pl.ds(r0, CHUNK)], sem)
    pltpu.async_copy(zsrc, acc.at[pl.ds(r0 + CHUNK, CHUNK)], sem)
    pltpu.async_copy(zsrc, acc.at[pl.ds(r0 + 2 * CHUNK, CHUNK)], sem)
    pltpu.async_copy(zsrc, acc.at[pl.ds(r0 + 3 * CHUNK, CHUNK)], sem)
    pltpu.async_copy(zsrc.at[pl.ds(0, RPS - 4 * CHUNK)],
                     acc.at[pl.ds(r0 + 4 * CHUNK, RPS - 4 * CHUNK)], sem)
    for _ in range(4):
        pltpu.make_async_copy(zsrc, acc.at[pl.ds(r0, CHUNK)], sem).wait()
    pltpu.make_async_copy(zsrc.at[pl.ds(0, RPS - 4 * CHUNK)],
                          acc.at[pl.ds(r0, RPS - 4 * CHUNK)], sem).wait()


def _unpack_chunk(packed, cc, sdst, ddst, b):
    for j in range(CHUNK // 16):
        p = packed[cc, pl.ds(j * 16, 16)]
        sdst[b, pl.ds(j * 16, 16)] = jax.lax.shift_right_logical(p, 14)
        ddst[b, pl.ds(j * 16, 16)] = jax.lax.bitwise_and(p, PACK - 1)


@functools.partial(
    pl.kernel,
    out_type=jax.ShapeDtypeStruct((NC, 2, ACC_ROWS, D), jnp.float32),
    mesh=_mesh,
    scratch_types=[
        pltpu.VMEM((NCHUNK, CHUNK), jnp.int32),
        pltpu.VMEM((NCHUNK, CHUNK), jnp.int32),
        pltpu.VMEM((NCHUNK, CHUNK), jnp.int32),
        pltpu.VMEM((CHUNK, D), jnp.float32),
        pltpu.VMEM_SHARED((ACC_ROWS, D), jnp.float32),
        pltpu.SemaphoreType.DMA,
        pltpu.SemaphoreType.DMA,
    ],
)
def _deg_kernel(edges_hbm, out_hbm, packed, sidx, didx, ones, acc, sem, zsem):
    c = lax.axis_index("c")
    s = lax.axis_index("s")
    wid = c * NS + s

    pltpu.sync_copy(edges_hbm.at[wid], packed)

    def unpack(cc, _):
        _unpack_chunk(packed, cc, sidx, didx, cc)
        return 0

    lax.fori_loop(0, NCHUNK, unpack, 0)

    r0 = s * RPS
    for which, idx in ((0, sidx), (1, didx)):
        _store_const(ones, CHUNK, 0.0)
        _zero_stripe(ones, acc, s, zsem)
        _store_const(ones, CHUNK, 1.0)
        plsc.subcore_barrier()

        def start(cc, _):
            pltpu.async_copy(ones, acc.at[idx.at[cc]], sem, add=True)
            return 0

        def drain(cc, _):
            pltpu.make_async_copy(ones, acc.at[idx.at[0]], sem).wait()
            return 0

        lax.fori_loop(0, NCHUNK, start, 0)
        lax.fori_loop(0, NCHUNK, drain, 0)
        plsc.subcore_barrier()
        pltpu.sync_copy(acc.at[pl.ds(r0, RPS)],
                        out_hbm.at[c, which, pl.ds(r0, RPS)])
        plsc.subcore_barrier()


@functools.partial(
    pl.kernel,
    out_type=jax.ShapeDtypeStruct((NC, ACC_ROWS, D), jnp.float32),
    mesh=_mesh,
    scratch_types=[
        pltpu.VMEM((CMAX, CHUNK), jnp.int32),
        pltpu.VMEM((2, CHUNK), jnp.int32),
        pltpu.VMEM((2, CHUNK), jnp.int32),
        pltpu.VMEM((2, CHUNK, D), jnp.float32),
        pltpu.VMEM_SHARED((ACC_ROWS, D), jnp.float32),
        pltpu.SemaphoreType.DMA,
        pltpu.SemaphoreType.DMA,
        pltpu.SemaphoreType.DMA,
    ],
)
def _msg_kernel(h_hbm, edges_hbm, out_hbm, packed, sbuf, dbuf, ring, acc,
                gsem0, gsem1, zsem):
    c = lax.axis_index("c")
    s = lax.axis_index("s")
    wid = c * NS + s
    gsems = (gsem0, gsem1)

    pltpu.sync_copy(edges_hbm.at[wid], packed)
    _store_const(ring.at[1], CHUNK, 0.0)
    _zero_stripe(ring.at[1], acc, s, zsem)
    plsc.subcore_barrier()

    for b in range(2):
        _unpack_chunk(packed, b, sbuf, dbuf, b)
        pltpu.async_copy(h_hbm.at[sbuf.at[b]], ring.at[b], gsems[b])

    def group(g, _):
        for b in range(2):
            cc = g * 2 + b
            pltpu.make_async_copy(h_hbm.at[sbuf.at[b]], ring.at[b],
                                  gsems[b]).wait()
            pltpu.sync_copy(ring.at[b], acc.at[dbuf.at[b]], add=True)
            _unpack_chunk(packed, cc + 2, sbuf, dbuf, b)
            pltpu.async_copy(h_hbm.at[sbuf.at[b]], ring.at[b], gsems[b])
        return 0

    ngrp = jnp.where(c == 0, C0 // 2, C1 // 2)
    lax.fori_loop(0, ngrp - 1, group, 0)
    for b in range(2):
        pltpu.make_async_copy(h_hbm.at[sbuf.at[b]], ring.at[b], gsems[b]).wait()
        pltpu.sync_copy(ring.at[b], acc.at[dbuf.at[b]], add=True)

    plsc.subcore_barrier()
    r0 = s * RPS
    pltpu.sync_copy(acc.at[pl.ds(r0, RPS)], out_hbm.at[c, pl.ds(r0, RPS)])


def _inv_sqrt_deg(cnt, which):
    c = cnt[0, which, 0:N, 0] + cnt[1, which, 0:N, 0]
    return lax.rsqrt(jnp.maximum(c, 1.0))


def _tc_pre_body(x_ref, w_ref, cnt_ref, out_ref):
    isd_out = _inv_sqrt_deg(cnt_ref[...], 0)
    h = jnp.dot(x_ref[...], w_ref[...], preferred_element_type=jnp.float32)
    out_ref[0:N, :] = h * isd_out[:, None]
    out_ref[N:H_ROWS, :] = jnp.zeros((H_ROWS - N, D), jnp.float32)


_tc_pre = pl.pallas_call(
    _tc_pre_body,
    out_shape=jax.ShapeDtypeStruct((H_ROWS, D), jnp.float32),
)


def _bn(h, gamma, beta):
    mu = jnp.mean(h, axis=0)
    var = jnp.mean((h - mu) ** 2, axis=0)
    return (h - mu) / jnp.sqrt(var + 1e-5) * gamma + beta


def _tc_mid_body(aggp_ref, cnt_ref, b1_ref, g1_ref, be1_ref, w2_ref,
                 h1_ref, out_ref):
    cnt = cnt_ref[...]
    isd_in = _inv_sqrt_deg(cnt, 1)
    agg = aggp_ref[0, 0:N, :] + aggp_ref[1, 0:N, :]
    h = agg * isd_in[:, None] + b1_ref[...]
    h = _bn(h, g1_ref[...], be1_ref[...])
    h1 = jnp.maximum(h, 0.0)
    h1_ref[...] = h1
    isd_out = _inv_sqrt_deg(cnt, 0)
    h2p = jnp.dot(h1, w2_ref[...], preferred_element_type=jnp.float32)
    out_ref[0:N, :] = h2p * isd_out[:, None]
    out_ref[N:H_ROWS, :] = jnp.zeros((H_ROWS - N, D), jnp.float32)


_tc_mid = pl.pallas_call(
    _tc_mid_body,
    out_shape=(
        jax.ShapeDtypeStruct((N, D), jnp.float32),
        jax.ShapeDtypeStruct((H_ROWS, D), jnp.float32),
    ),
)


def _tc_post_body(aggp_ref, cnt_ref, b2_ref, g2_ref, be2_ref, h1_ref, out_ref):
    isd_in = _inv_sqrt_deg(cnt_ref[...], 1)
    agg = aggp_ref[0, 0:N, :] + aggp_ref[1, 0:N, :]
    h = agg * isd_in[:, None] + b2_ref[...]
    h = _bn(h, g2_ref[...], be2_ref[...])
    out_ref[...] = jnp.maximum(h + h1_ref[...], 0.0)


_tc_post = pl.pallas_call(
    _tc_post_body,
    out_shape=jax.ShapeDtypeStruct((N, D), jnp.float32),
)


def kernel(x, edge_index, W1, b1, gamma1, beta1, W2, b2, gamma2, beta2):
    src = edge_index[0].astype(jnp.int32)
    dst = edge_index[1].astype(jnp.int32)
    packed = src * PACK + dst
    pad = jnp.full((E_PAD - E,), PAD_EDGE, jnp.int32)
    flat = jnp.concatenate([packed, pad])
    edges_deg = flat.reshape(NW, NCHUNK, CHUNK)
    a = flat[: C0 * NS * CHUNK].reshape(NS, C0, CHUNK)
    a = jnp.concatenate(
        [a, jnp.full((NS, CMAX - C0, CHUNK), PAD_EDGE, jnp.int32)], axis=1)
    b = flat[C0 * NS * CHUNK:].reshape(NS, C1, CHUNK)
    edges_msg = jnp.concatenate([a, b], axis=0)

    cnt = _deg_kernel(edges_deg)
    h1pre = _tc_pre(x, W1, cnt)
    agg1 = _msg_kernel(h1pre, edges_msg)
    h1, h2pre = _tc_mid(agg1, cnt, b1, gamma1, beta1, W2)
    agg2 = _msg_kernel(h2pre, edges_msg)
    return _tc_post(agg2, cnt, b2, gamma2, beta2, h1)


# speedup vs baseline: 1.3683x; 1.3683x over previous
import functools

import jax
import jax.numpy as jnp
from jax import lax
from jax.experimental import pallas as pl
from jax.experimental.pallas import tpu as pltpu
from jax.experimental.pallas import tpu_sc as plsc

N = 10000
E = 320000
D = 128

NC = 2
NS = 16
NW = NC * NS
CHUNK = 128
NCHUNK = 80
EPW = NCHUNK * CHUNK
E_PAD = EPW * NW
H_ROWS = N + 16
ACC_ROWS = 10112
RPS = ACC_ROWS // NS
PACK = 1 << 14
PAD_EDGE = N * PACK + N
C0 = 32
C1 = 2 * NCHUNK - C0
CMAX = max(C0, C1)

_mesh = plsc.VectorSubcoreMesh(core_axis_name="c", subcore_axis_name="s")


def _store_const(buf, rows, value, dtype=jnp.float32):
    vv = jnp.full((16,), value, dtype)

    def body(i, _):
        for j in range(D // 16):
            buf[i, pl.ds(j * 16, 16)] = vv
        return 0

    lax.fori_loop(0, rows, body, 0)


def _zero_stripe(zsrc, acc, s, sem):
    r0 = s * RPS
    pltpu.async_copy(zsrc, acc.at[pl.ds(r0, CHUNK)], sem)
    pltpu.async_copy(zsrc, acc.at[pl.ds(r0 + CHUNK, CHUNK)], sem)
    pltpu.async_copy(zsrc, acc.at[pl.ds(r0 + 2 * CHUNK, CHUNK)], sem)
    pltpu.async_copy(zsrc, acc.at[pl.ds(r0 + 3 * CHUNK, CHUNK)], sem)
    pltpu.async_copy(zsrc.at[pl.ds(0, RPS - 4 * CHUNK)],
                     acc.at[pl.ds(r0 + 4 * CHUNK, RPS - 4 * CHUNK)], sem)
    for _ in range(4):
        pltpu.make_async_copy(zsrc, acc.at[pl.ds(r0, CHUNK)], sem).wait()
    pltpu.make_async_copy(zsrc.at[pl.ds(0, RPS - 4 * CHUNK)],
                          acc.at[pl.ds(r0, RPS - 4 * CHUNK)], sem).wait()


def _unpack_chunk(packed, cc, sdst, ddst, b):
    for j in range(CHUNK // 16):
        p = packed[cc, pl.ds(j * 16, 16)]
        sdst[b, pl.ds(j * 16, 16)] = jax.lax.shift_right_logical(p, 14)
        ddst[b, pl.ds(j * 16, 16)] = jax.lax.bitwise_and(p, PACK - 1)


@functools.partial(
    pl.kernel,
    out_type=jax.ShapeDtypeStruct((NC, 2, ACC_ROWS, D), jnp.float32),
    mesh=_mesh,
    scratch_types=[
        pltpu.VMEM((NCHUNK, CHUNK), jnp.int32),
        pltpu.VMEM((NCHUNK, CHUNK), jnp.int32),
        pltpu.VMEM((NCHUNK, CHUNK), jnp.int32),
        pltpu.VMEM((CHUNK, D), jnp.float32),
        pltpu.VMEM_SHARED((ACC_ROWS, D), jnp.float32),
        pltpu.SemaphoreType.DMA,
        pltpu.SemaphoreType.DMA,
    ],
)
def _deg_kernel(edges_hbm, out_hbm, packed, sidx, didx, ones, acc, sem, zsem):
    c = lax.axis_index("c")
    s = lax.axis_index("s")
    wid = c * NS + s

    pltpu.sync_copy(edges_hbm.at[wid], packed)

    def unpack(cc, _):
        _unpack_chunk(packed, cc, sidx, didx, cc)
        return 0

    lax.fori_loop(0, NCHUNK, unpack, 0)

    r0 = s * RPS
    for which, idx in ((0, sidx), (1, didx)):
        _store_const(ones, CHUNK, 0.0)
        _zero_stripe(ones, acc, s, zsem)
        _store_const(ones, CHUNK, 1.0)
        plsc.subcore_barrier()

        def start(cc, _):
            pltpu.async_copy(ones, acc.at[idx.at[cc]], sem, add=True)
            return 0

        def drain(cc, _):
            pltpu.make_async_copy(ones, acc.at[idx.at[0]], sem).wait()
            return 0

        lax.fori_loop(0, NCHUNK, start, 0)
        lax.fori_loop(0, NCHUNK, drain, 0)
        plsc.subcore_barrier()
        pltpu.sync_copy(acc.at[pl.ds(r0, RPS)],
                        out_hbm.at[c, which, pl.ds(r0, RPS)])
        plsc.subcore_barrier()


@functools.partial(
    pl.kernel,
    out_type=jax.ShapeDtypeStruct((NC, ACC_ROWS, D), jnp.float32),
    mesh=_mesh,
    scratch_types=[
        pltpu.VMEM((CMAX, CHUNK), jnp.int32),
        pltpu.VMEM((2, CHUNK), jnp.int32),
        pltpu.VMEM((2, CHUNK), jnp.int32),
        pltpu.VMEM((2, CHUNK, D), jnp.float32),
        pltpu.VMEM_SHARED((ACC_ROWS, D), jnp.float32),
        pltpu.SemaphoreType.DMA,
        pltpu.SemaphoreType.DMA,
        pltpu.SemaphoreType.DMA,
    ],
)
def _msg_kernel(h_hbm, edges_hbm, out_hbm, packed, sbuf, dbuf, ring, acc,
                gsem0, gsem1, zsem):
    c = lax.axis_index("c")
    s = lax.axis_index("s")
    wid = c * NS + s
    gsems = (gsem0, gsem1)

    pltpu.sync_copy(edges_hbm.at[wid], packed)
    _store_const(ring.at[1], CHUNK, 0.0)
    _zero_stripe(ring.at[1], acc, s, zsem)
    plsc.subcore_barrier()

    for b in range(2):
        _unpack_chunk(packed, b, sbuf, dbuf, b)
        pltpu.async_copy(h_hbm.at[sbuf.at[b]], ring.at[b], gsems[b])

    def group(g, _):
        for b in range(2):
            cc = g * 2 + b
            pltpu.make_async_copy(h_hbm.at[sbuf.at[b]], ring.at[b],
                                  gsems[b]).wait()
            pltpu.sync_copy(ring.at[b], acc.at[dbuf.at[b]], add=True)
            _unpack_chunk(packed, cc + 2, sbuf, dbuf, b)
            pltpu.async_copy(h_hbm.at[sbuf.at[b]], ring.at[b], gsems[b])
        return 0

    ngrp = jnp.where(c == 0, C0 // 2, C1 // 2)
    lax.fori_loop(0, ngrp - 1, group, 0)
    for b in range(2):
        pltpu.make_async_copy(h_hbm.at[sbuf.at[b]], ring.at[b], gsems[b]).wait()
        pltpu.sync_copy(ring.at[b], acc.at[dbuf.at[b]], add=True)

    plsc.subcore_barrier()
    r0 = s * RPS
    pltpu.sync_copy(acc.at[pl.ds(r0, RPS)], out_hbm.at[c, pl.ds(r0, RPS)])


def _inv_sqrt_deg(cnt, which):
    c = cnt[0, which, 0:N, 0] + cnt[1, which, 0:N, 0]
    return lax.rsqrt(jnp.maximum(c, 1.0))


def _tc_pre_body(x_ref, w_ref, cnt_ref, out_ref):
    isd_out = _inv_sqrt_deg(cnt_ref[...], 0)
    h = jnp.dot(x_ref[...], w_ref[...], preferred_element_type=jnp.float32)
    out_ref[0:N, :] = h * isd_out[:, None]
    out_ref[N:H_ROWS, :] = jnp.zeros((H_ROWS - N, D), jnp.float32)


_tc_pre = pl.pallas_call(
    _tc_pre_body,
    out_shape=jax.ShapeDtypeStruct((H_ROWS, D), jnp.float32),
)


def _bn(h, gamma, beta):
    mu = jnp.mean(h, axis=0)
    var = jnp.mean((h - mu) ** 2, axis=0)
    return (h - mu) / jnp.sqrt(var + 1e-5) * gamma + beta


def _tc_mid_body(aggp_ref, cnt_ref, b1_ref, g1_ref, be1_ref, w2_ref,
                 h1_ref, out_ref):
    cnt = cnt_ref[...]
    isd_in = _inv_sqrt_deg(cnt, 1)
    agg = aggp_ref[0, 0:N, :] + aggp_ref[1, 0:N, :]
    h = agg * isd_in[:, None] + b1_ref[...]
    h = _bn(h, g1_ref[...], be1_ref[...])
    h1 = jnp.maximum(h, 0.0)
    h1_ref[...] = h1
    isd_out = _inv_sqrt_deg(cnt, 0)
    h2p = jnp.dot(h1, w2_ref[...], preferred_element_type=jnp.float32)
    out_ref[0:N, :] = h2p * isd_out[:, None]
    out_ref[N:H_ROWS, :] = jnp.zeros((H_ROWS - N, D), jnp.float32)


_tc_mid = pl.pallas_call(
    _tc_mid_body,
    out_shape=(
        jax.ShapeDtypeStruct((N, D), jnp.float32),
        jax.ShapeDtypeStruct((H_ROWS, D), jnp.float32),
    ),
)


def _tc_post_body(aggp_ref, cnt_ref, b2_ref, g2_ref, be2_ref, h1_ref, out_ref):
    isd_in = _inv_sqrt_deg(cnt_ref[...], 1)
    agg = aggp_ref[0, 0:N, :] + aggp_ref[1, 0:N, :]
    h = agg * isd_in[:, None] + b2_ref[...]
    h = _bn(h, g2_ref[...], be2_ref[...])
    out_ref[...] = jnp.maximum(h + h1_ref[...], 0.0)


_tc_post = pl.pallas_call(
    _tc_post_body,
    out_shape=jax.ShapeDtypeStruct((N, D), jnp.float32),
)


def kernel(x, edge_index, W1, b1, gamma1, beta1, W2, b2, gamma2, beta2):
    src = edge_index[0].astype(jnp.int32)
    dst = edge_index[1].astype(jnp.int32)
    packed = src * PACK + dst
    pad = jnp.full((E_PAD - E,), PAD_EDGE, jnp.int32)
    flat = jnp.concatenate([packed, pad])
    edges_deg = flat.reshape(NW, NCHUNK, CHUNK)
    a = flat[: C0 * NS * CHUNK].reshape(NS, C0, CHUNK)
    a = jnp.concatenate(
        [a, jnp.full((NS, CMAX - C0, CHUNK), PAD_EDGE, jnp.int32)], axis=1)
    b = flat[C0 * NS * CHUNK:].reshape(NS, C1, CHUNK)
    edges_msg = jnp.concatenate([a, b], axis=0)

    cnt = _deg_kernel(edges_deg)
    h1pre = _tc_pre(x, W1, cnt)
    agg1 = _msg_kernel(h1pre, edges_msg)
    h1, h2pre = _tc_mid(agg1, cnt, b1, gamma1, beta1, W2)
    agg2 = _msg_kernel(h2pre, edges_msg)
    return _tc_post(agg2, cnt, b2, gamma2, beta2, h1)
